# baseline (speedup 1.0000x reference)
import functools

import jax
import jax.numpy as jnp
from jax import lax
from jax.experimental import pallas as pl
from jax.experimental.pallas import tpu as pltpu
from jax.experimental.pallas import tpu_sc as plsc

NV = 6890
P = 1024
W32 = (NV * NV) // 4
L = 16
NCHUNK = P // 128


def _sc_gather(pc, verts, gm32):
    info = plsc.get_sparse_core_info()
    nw = info.num_cores * info.num_subcores
    rpw = P // nw

    mesh = plsc.VectorSubcoreMesh(core_axis_name="c", subcore_axis_name="s")

    @functools.partial(
        pl.kernel,
        mesh=mesh,
        out_type=[
            jax.ShapeDtypeStruct((P * 3,), jnp.float32),
            jax.ShapeDtypeStruct((3, P), jnp.float32),
            jax.ShapeDtypeStruct((P, P), jnp.int32),
        ],
        scratch_types=[
            pltpu.VMEM((P,), jnp.int32),
            pltpu.VMEM((NV * 3,), jnp.float32),
            pltpu.VMEM((rpw * 3,), jnp.float32),
            pltpu.VMEM((3, rpw), jnp.float32),
            pltpu.VMEM((NCHUNK, 128), jnp.int32),
            pltpu.VMEM((NCHUNK, 128), jnp.int32),
            pltpu.VMEM((P,), jnp.int32),
            pltpu.SemaphoreType.DMA,
        ],
        compiler_params=pltpu.CompilerParams(needs_layout_passes=False),
    )
    def sc_fn(pc_hbm, verts_hbm, gm32_hbm, vpr_hbm, vpt_hbm, mg_hbm,
              pc_v, verts_v, vpr_v, vpt_v, idx_v, wbuf_v, row_v, sem):
        wid = lax.axis_index("s") * info.num_cores + lax.axis_index("c")
        base = wid * rpw

        pltpu.sync_copy(pc_hbm, pc_v)
        pltpu.sync_copy(verts_hbm, verts_v)

        lane = lax.iota(jnp.int32, L)
        for k in range(rpw // L):
            rvec = pc_v[pl.ds(base + k * L, L)]
            for c in range(3):
                val = plsc.load_gather(verts_v, [rvec * 3 + c])
                vpt_v[c, pl.ds(k * L, L)] = val
                plsc.store_scatter(vpr_v, [(lane + k * L) * 3 + c], val)
        pltpu.sync_copy(vpr_v, vpr_hbm.at[pl.ds(base * 3, rpw * 3)])
        for c in range(3):
            pltpu.sync_copy(vpt_v.at[c], vpt_hbm.at[c, pl.ds(base, rpw)])

        def body(i, carry):
            rb = plsc.load_gather(pc_v, [jnp.full((L,), base + i, jnp.int32)])
            rbase = rb * NV
            for k in range(NCHUNK):
                for v in range(128 // L):
                    pcv = pc_v[pl.ds(k * 128 + v * L, L)]
                    bidx = rbase + pcv
                    idx_v[k, pl.ds(v * L, L)] = bidx >> 2
            copies = []
            for k in range(NCHUNK):
                copies.append(pltpu.async_copy(
                    gm32_hbm.at[idx_v.at[k]], wbuf_v.at[k], sem))
            for cp in copies:
                cp.wait()
            for k in range(NCHUNK):
                for v in range(128 // L):
                    pcv = pc_v[pl.ds(k * 128 + v * L, L)]
                    bidx = rbase + pcv
                    w = wbuf_v[k, pl.ds(v * L, L)]
                    sh = (bidx & 3) * 8
                    bit = lax.shift_right_logical(w, sh) & 1
                    row_v[pl.ds(k * 128 + v * L, L)] = bit
            pltpu.sync_copy(row_v, mg_hbm.at[base + i])
            return carry

        lax.fori_loop(0, rpw, body, 0)

    return sc_fn(pc, verts, gm32)


def _tc_loss(vpr, vpt, mg):
    def tc_fn(vpr_ref, vpt_ref, mg_ref, out_ref):
        s = jnp.zeros((P, P), jnp.float32)
        for c in range(3):
            col = vpr_ref[:, c:c + 1]
            row = vpt_ref[c:c + 1, :]
            d = col - row
            s = s + d * d
        big = jnp.float32(3.0e37)
        sm = jnp.where(mg_ref[...] > 0, s, big)
        rmin = jnp.min(sm, axis=1, keepdims=True)
        rmin = jnp.where(rmin >= big * 0.5, s[:, 0:1], rmin)
        loss = jnp.mean(jnp.tanh(jnp.sqrt(rmin)))
        out_ref[0, 0] = loss

    out = pl.pallas_call(
        tc_fn,
        out_shape=jax.ShapeDtypeStruct((1, 1), jnp.float32),
        out_specs=pl.BlockSpec(memory_space=pltpu.SMEM),
    )(vpr, vpt, mg)
    return out[0, 0]


def kernel(presented_contact, vertices, geomask):
    pc = presented_contact.astype(jnp.int32)
    verts = vertices.reshape(NV * 3)
    gm32 = lax.bitcast_convert_type(
        geomask.astype(jnp.uint8).reshape(W32, 4), jnp.int32)
    vpr_flat, vpt, mg = _sc_gather(pc, verts, gm32)
    return _tc_loss(vpr_flat.reshape(P, 3), vpt, mg)


# speedup vs baseline: 9.2662x; 9.2662x over previous
import functools

import jax
import jax.numpy as jnp
from jax import lax
from jax.experimental import pallas as pl
from jax.experimental.pallas import tpu as pltpu
from jax.experimental.pallas import tpu_sc as plsc

NV = 6890
P = 1024
NVP = 7168
NW32 = NVP // 4
VW = 128
KT = 896
NKT = NVP // KT


def _sc_gather(pc, gmw, vpad):
    info = plsc.get_sparse_core_info()
    nw = info.num_cores * info.num_subcores
    rpw = P // nw

    mesh = plsc.VectorSubcoreMesh(core_axis_name="c", subcore_axis_name="s")

    @functools.partial(
        pl.kernel,
        mesh=mesh,
        out_type=[
            jax.ShapeDtypeStruct((P, NW32), jnp.int32),
            jax.ShapeDtypeStruct((P, VW), jnp.float32),
        ],
        scratch_types=[
            pltpu.VMEM((rpw,), jnp.int32),
            pltpu.VMEM((rpw, NW32), jnp.int32),
            pltpu.VMEM((rpw, VW), jnp.float32),
            pltpu.SemaphoreType.DMA,
        ],
    )
    def sc_fn(pc_hbm, gmw_hbm, vpad_hbm, grow_hbm, vpg_hbm,
              idx_v, rows_v, vrows_v, sem):
        wid = lax.axis_index("s") * info.num_cores + lax.axis_index("c")
        base = wid * rpw
        pltpu.sync_copy(pc_hbm.at[pl.ds(base, rpw)], idx_v)
        cp1 = pltpu.async_copy(gmw_hbm.at[idx_v], rows_v, sem)
        cp2 = pltpu.async_copy(vpad_hbm.at[idx_v], vrows_v, sem)
        cp1.wait()
        cp2.wait()
        pltpu.sync_copy(rows_v, grow_hbm.at[pl.ds(base, rpw)])
        pltpu.sync_copy(vrows_v, vpg_hbm.at[pl.ds(base, rpw)])

    return sc_fn(pc, gmw, vpad)


def _tc_loss(pc_row, vpg, grow8):
    def tc_fn(pc_ref, vp_ref, g8_ref, out_ref, acc_ref):
        kt = pl.program_id(0)

        @pl.when(kt == 0)
        def _():
            acc_ref[...] = jnp.zeros((P, P), jnp.float32)

        kio = lax.broadcasted_iota(jnp.int32, (KT, P), 0) + kt * KT
        oh = (kio == pc_ref[...]).astype(jnp.bfloat16)
        acc_ref[...] += jnp.dot(g8_ref[...].astype(jnp.bfloat16), oh,
                                preferred_element_type=jnp.float32)

        @pl.when(kt == NKT - 1)
        def _():
            vp = vp_ref[...]
            s = jnp.zeros((P, P), jnp.float32)
            for c in range(3):
                col = vp[:, c:c + 1]
                e = (lax.broadcasted_iota(jnp.int32, (1, VW), 1) == c
                     ).astype(jnp.float32)
                row = lax.dot_general(e, vp, (((1,), (1,)), ((), ())),
                                      preferred_element_type=jnp.float32)
                d = col - row
                s = s + d * d
            big = jnp.float32(3.0e37)
            sm = jnp.where(acc_ref[...] > 0.5, s, big)
            rmin = jnp.min(sm, axis=1, keepdims=True)
            rmin = jnp.where(rmin >= big * 0.5, s[:, 0:1], rmin)
            out_ref[0, 0] = jnp.mean(jnp.tanh(jnp.sqrt(rmin)))

    out = pl.pallas_call(
        tc_fn,
        grid=(NKT,),
        in_specs=[
            pl.BlockSpec((1, P), lambda kt: (0, 0)),
            pl.BlockSpec((P, VW), lambda kt: (0, 0)),
            pl.BlockSpec((P, KT), lambda kt: (0, kt)),
        ],
        out_specs=pl.BlockSpec(memory_space=pltpu.SMEM),
        out_shape=jax.ShapeDtypeStruct((1, 1), jnp.float32),
        scratch_shapes=[pltpu.VMEM((P, P), jnp.float32)],
    )(pc_row, vpg, grow8)
    return out[0, 0]


def kernel(presented_contact, vertices, geomask):
    pc = presented_contact.astype(jnp.int32)
    v = vertices[0]
    vpad = jnp.pad(v, ((0, 0), (0, VW - 3)))
    gmp = jnp.pad(geomask.astype(jnp.uint8), ((0, 0), (0, NVP - NV)))
    gmw = lax.bitcast_convert_type(gmp.reshape(NV, NW32, 4), jnp.int32)
    grow, vpg = _sc_gather(pc, gmw, vpad)
    grow8 = lax.bitcast_convert_type(grow, jnp.uint8).reshape(P, NVP)
    return _tc_loss(pc.reshape(1, P), vpg, grow8)


# speedup vs baseline: 33.2814x; 3.5917x over previous
import functools

import jax
import jax.numpy as jnp
from jax import lax
from jax.experimental import pallas as pl
from jax.experimental.pallas import tpu as pltpu
from jax.experimental.pallas import tpu_sc as plsc

NV = 6890
P = 1024
NVP = 6912
NQ = NVP // 4
VW = 128
KT = 1152
NKT = NVP // KT
WRB = 216
NWRB = NQ // WRB


def _repack(gm):
    def rp_fn(gm_ref, gmq_ref):
        x = gm_ref[...].astype(jnp.uint8)
        xp = jnp.pad(x, ((0, 0), (0, NVP - NV)))
        gmq_ref[...] = pltpu.bitcast(xp, jnp.int32)

    return pl.pallas_call(
        rp_fn,
        grid=(NWRB,),
        in_specs=[pl.BlockSpec((4 * WRB, NV), lambda r: (r, 0))],
        out_specs=pl.BlockSpec((WRB, NVP), lambda r: (r, 0)),
        out_shape=jax.ShapeDtypeStruct((NQ, NVP), jnp.int32),
    )(gm)


def _vpad(v):
    def vp_fn(v_ref, vp_ref):
        vp_ref[...] = jnp.pad(v_ref[...], ((0, 0), (0, VW - 3)))

    return pl.pallas_call(
        vp_fn,
        out_shape=jax.ShapeDtypeStruct((NV, VW), jnp.float32),
    )(v)


def _sc_gather(pc, pcq, gmq, vpad):
    info = plsc.get_sparse_core_info()
    nw = info.num_cores * info.num_subcores
    rpw = P // nw
    half = rpw // 2

    mesh = plsc.VectorSubcoreMesh(core_axis_name="c", subcore_axis_name="s")

    @functools.partial(
        pl.kernel,
        mesh=mesh,
        out_type=[
            jax.ShapeDtypeStruct((P, NVP), jnp.int32),
            jax.ShapeDtypeStruct((P, VW), jnp.float32),
        ],
        scratch_types=[
            pltpu.VMEM((rpw,), jnp.int32),
            pltpu.VMEM((half,), jnp.int32),
            pltpu.VMEM((half, NVP), jnp.int32),
            pltpu.VMEM((rpw, VW), jnp.float32),
            pltpu.SemaphoreType.DMA,
            pltpu.SemaphoreType.DMA,
        ],
    )
    def sc_fn(pc_hbm, pcq_hbm, gmq_hbm, vpad_hbm, grow_hbm, vpg_hbm,
              vidx_v, idx_v, rows_v, vrows_v, vsem, sem):
        wid = lax.axis_index("s") * info.num_cores + lax.axis_index("c")
        base = wid * rpw
        pltpu.sync_copy(pc_hbm.at[pl.ds(base, rpw)], vidx_v)
        cpv = pltpu.async_copy(vpad_hbm.at[vidx_v], vrows_v, vsem)
        for ch in range(2):
            pltpu.sync_copy(pcq_hbm.at[pl.ds(base + ch * half, half)], idx_v)
            pltpu.async_copy(gmq_hbm.at[idx_v], rows_v, sem).wait()
            pltpu.sync_copy(rows_v, grow_hbm.at[pl.ds(base + ch * half, half)])
        cpv.wait()
        pltpu.sync_copy(vrows_v, vpg_hbm.at[pl.ds(base, rpw)])

    return sc_fn(pc, pcq, gmq, vpad)


def _tc_loss(pc_row, psh_col, vpg, grow):
    def tc_fn(pc_ref, psh_ref, vp_ref, g_ref, out_ref, acc_ref):
        kt = pl.program_id(0)

        @pl.when(kt == 0)
        def _():
            acc_ref[...] = jnp.zeros((P, P), jnp.float32)

        w = g_ref[...]
        ext = (w >> psh_ref[...]) & 1
        kio = lax.broadcasted_iota(jnp.int32, (KT, P), 0) + kt * KT
        oh = (kio == pc_ref[...]).astype(jnp.bfloat16)
        acc_ref[...] += jnp.dot(ext.astype(jnp.bfloat16), oh,
                                preferred_element_type=jnp.float32)

        @pl.when(kt == NKT - 1)
        def _():
            vp = vp_ref[...]
            s = jnp.zeros((P, P), jnp.float32)
            for c in range(3):
                col = vp[:, c:c + 1]
                e = (lax.broadcasted_iota(jnp.int32, (1, VW), 1) == c
                     ).astype(jnp.float32)
                row = lax.dot_general(e, vp, (((1,), (1,)), ((), ())),
                                      preferred_element_type=jnp.float32)
                d = col - row
                s = s + d * d
            big = jnp.float32(3.0e37)
            sm = jnp.where(acc_ref[...] > 0.5, s, big)
            rmin = jnp.min(sm, axis=1, keepdims=True)
            rmin = jnp.where(rmin >= big * 0.5, s[:, 0:1], rmin)
            out_ref[0, 0] = jnp.mean(jnp.tanh(jnp.sqrt(rmin)))

    out = pl.pallas_call(
        tc_fn,
        grid=(NKT,),
        in_specs=[
            pl.BlockSpec((1, P), lambda kt: (0, 0)),
            pl.BlockSpec((P, 1), lambda kt: (0, 0)),
            pl.BlockSpec((P, VW), lambda kt: (0, 0)),
            pl.BlockSpec((P, KT), lambda kt: (0, kt)),
        ],
        out_specs=pl.BlockSpec(memory_space=pltpu.SMEM),
        out_shape=jax.ShapeDtypeStruct((1, 1), jnp.float32),
        scratch_shapes=[pltpu.VMEM((P, P), jnp.float32)],
    )(pc_row, psh_col, vpg, grow)
    return out[0, 0]


def kernel(presented_contact, vertices, geomask):
    pc = presented_contact.astype(jnp.int32)
    gmq = _repack(geomask)
    vpad = _vpad(vertices[0])
    grow, vpg = _sc_gather(pc, pc >> 2, gmq, vpad)
    psh = ((pc & 3) * 8).reshape(P, 1)
    return _tc_loss(pc.reshape(1, P), psh, vpg, grow)


# speedup vs baseline: 54.8525x; 1.6481x over previous
import functools

import jax
import jax.numpy as jnp
from jax import lax
from jax.experimental import pallas as pl
from jax.experimental.pallas import tpu as pltpu
from jax.experimental.pallas import tpu_sc as plsc

NV = 6890
P = 1024
NVP = 6912
NQ = NVP // 4
VW = 128
KT = 1152
NKT = NVP // KT
WRB = 96
NWRB = NQ // WRB


def _repack(gm):
    def rp_fn(gm_ref, gmq_ref):
        x = gm_ref[...]
        xp = jnp.pad(x, ((0, 0), (0, NVP - NV)))
        gmq_ref[...] = pltpu.bitcast(xp, jnp.int32)

    return pl.pallas_call(
        rp_fn,
        grid=(NWRB,),
        in_specs=[pl.BlockSpec((4 * WRB, NV), lambda r: (r, 0))],
        out_specs=pl.BlockSpec((WRB, NVP), lambda r: (r, 0)),
        out_shape=jax.ShapeDtypeStruct((NQ, NVP), jnp.int32),
    )(gm)


def _vpad(v):
    def vp_fn(v_ref, vp_ref):
        vp_ref[...] = jnp.pad(v_ref[...], ((0, 0), (0, VW - 3)))

    return pl.pallas_call(
        vp_fn,
        out_shape=jax.ShapeDtypeStruct((NV, VW), jnp.float32),
    )(v)


def _sc_gather(pc, pcq, gmq, vpad):
    info = plsc.get_sparse_core_info()
    nw = info.num_cores * info.num_subcores
    rpw = P // nw
    half = rpw // 2

    mesh = plsc.VectorSubcoreMesh(core_axis_name="c", subcore_axis_name="s")

    @functools.partial(
        pl.kernel,
        mesh=mesh,
        out_type=[
            jax.ShapeDtypeStruct((P, NVP), jnp.int32),
            jax.ShapeDtypeStruct((P, VW), jnp.float32),
        ],
        scratch_types=[
            pltpu.VMEM((rpw,), jnp.int32),
            pltpu.VMEM((half,), jnp.int32),
            pltpu.VMEM((half, NVP), jnp.int32),
            pltpu.VMEM((rpw, VW), jnp.float32),
            pltpu.SemaphoreType.DMA,
            pltpu.SemaphoreType.DMA,
        ],
    )
    def sc_fn(pc_hbm, pcq_hbm, gmq_hbm, vpad_hbm, grow_hbm, vpg_hbm,
              vidx_v, idx_v, rows_v, vrows_v, vsem, sem):
        wid = lax.axis_index("s") * info.num_cores + lax.axis_index("c")
        base = wid * rpw
        pltpu.sync_copy(pc_hbm.at[pl.ds(base, rpw)], vidx_v)
        cpv = pltpu.async_copy(vpad_hbm.at[vidx_v], vrows_v, vsem)
        for ch in range(2):
            pltpu.sync_copy(pcq_hbm.at[pl.ds(base + ch * half, half)], idx_v)
            pltpu.async_copy(gmq_hbm.at[idx_v], rows_v, sem).wait()
            pltpu.sync_copy(rows_v, grow_hbm.at[pl.ds(base + ch * half, half)])
        cpv.wait()
        pltpu.sync_copy(vrows_v, vpg_hbm.at[pl.ds(base, rpw)])

    return sc_fn(pc, pcq, gmq, vpad)


def _tc_loss(pc_row, psh_col, vpg, grow):
    def tc_fn(pc_ref, psh_ref, vp_ref, g_ref, out_ref, acc_ref):
        kt = pl.program_id(0)

        @pl.when(kt == 0)
        def _():
            acc_ref[...] = jnp.zeros((P, P), jnp.float32)

        w = g_ref[...]
        ext = (w >> psh_ref[...]) & 1
        kio = lax.broadcasted_iota(jnp.int32, (KT, P), 0) + kt * KT
        oh = (kio == pc_ref[...]).astype(jnp.bfloat16)
        acc_ref[...] += jnp.dot(ext.astype(jnp.bfloat16), oh,
                                preferred_element_type=jnp.float32)

        @pl.when(kt == NKT - 1)
        def _():
            vp = vp_ref[...]
            s = jnp.zeros((P, P), jnp.float32)
            for c in range(3):
                col = vp[:, c:c + 1]
                e = (lax.broadcasted_iota(jnp.int32, (1, VW), 1) == c
                     ).astype(jnp.float32)
                row = lax.dot_general(e, vp, (((1,), (1,)), ((), ())),
                                      preferred_element_type=jnp.float32)
                d = col - row
                s = s + d * d
            big = jnp.float32(3.0e37)
            sm = jnp.where(acc_ref[...] > 0.5, s, big)
            rmin = jnp.min(sm, axis=1, keepdims=True)
            rmin = jnp.where(rmin >= big * 0.5, s[:, 0:1], rmin)
            out_ref[0, 0] = jnp.mean(jnp.tanh(jnp.sqrt(rmin)))

    out = pl.pallas_call(
        tc_fn,
        grid=(NKT,),
        in_specs=[
            pl.BlockSpec((1, P), lambda kt: (0, 0)),
            pl.BlockSpec((P, 1), lambda kt: (0, 0)),
            pl.BlockSpec((P, VW), lambda kt: (0, 0)),
            pl.BlockSpec((P, KT), lambda kt: (0, kt)),
        ],
        out_specs=pl.BlockSpec(memory_space=pltpu.SMEM),
        out_shape=jax.ShapeDtypeStruct((1, 1), jnp.float32),
        scratch_shapes=[pltpu.VMEM((P, P), jnp.float32)],
    )(pc_row, psh_col, vpg, grow)
    return out[0, 0]


def kernel(presented_contact, vertices, geomask):
    pc = presented_contact.astype(jnp.int32)
    gm8 = geomask.view(jnp.uint8)
    gmq = _repack(gm8)
    vpad = _vpad(vertices[0])
    grow, vpg = _sc_gather(pc, pc >> 2, gmq, vpad)
    psh = ((pc & 3) * 8).reshape(P, 1)
    return _tc_loss(pc.reshape(1, P), psh, vpg, grow)


# speedup vs baseline: 55.1385x; 1.0052x over previous
import functools

import jax
import jax.numpy as jnp
from jax import lax
from jax.experimental import pallas as pl
from jax.experimental.pallas import tpu as pltpu
from jax.experimental.pallas import tpu_sc as plsc

NV = 6890
P = 1024
NVP = 6912
NQ = NVP // 4
VW = 128
KT = 1152
NKT = NVP // KT
WRB = 96
NWRB = NQ // WRB


def _repack(gm, v):
    def rp_fn(gm_ref, v_ref, gmq_ref, vp_ref):
        x = gm_ref[...]
        xp = jnp.pad(x, ((0, 0), (0, NVP - NV)))
        gmq_ref[...] = pltpu.bitcast(xp, jnp.int32)

        @pl.when(pl.program_id(0) == 0)
        def _():
            vp_ref[...] = jnp.pad(v_ref[...], ((0, 0), (0, VW - 3)))

    return pl.pallas_call(
        rp_fn,
        grid=(NWRB,),
        in_specs=[
            pl.BlockSpec((4 * WRB, NV), lambda r: (r, 0)),
            pl.BlockSpec((NV, 3), lambda r: (0, 0)),
        ],
        out_specs=[
            pl.BlockSpec((WRB, NVP), lambda r: (r, 0)),
            pl.BlockSpec((NV, VW), lambda r: (0, 0)),
        ],
        out_shape=[
            jax.ShapeDtypeStruct((NQ, NVP), jnp.int32),
            jax.ShapeDtypeStruct((NV, VW), jnp.float32),
        ],
    )(gm, v)


def _sc_gather(pc, pcq, gmq, vpad):
    info = plsc.get_sparse_core_info()
    nw = info.num_cores * info.num_subcores
    rpw = P // nw
    qtr = rpw // 4
    nch = 4

    mesh = plsc.VectorSubcoreMesh(core_axis_name="c", subcore_axis_name="s")

    @functools.partial(
        pl.kernel,
        mesh=mesh,
        out_type=[
            jax.ShapeDtypeStruct((P, NVP), jnp.int32),
            jax.ShapeDtypeStruct((P, VW), jnp.float32),
        ],
        scratch_types=[
            pltpu.VMEM((rpw,), jnp.int32),
            pltpu.VMEM((nch, qtr), jnp.int32),
            pltpu.VMEM((2, qtr, NVP), jnp.int32),
            pltpu.VMEM((rpw, VW), jnp.float32),
            pltpu.SemaphoreType.DMA,
            pltpu.SemaphoreType.DMA,
        ],
    )
    def sc_fn(pc_hbm, pcq_hbm, gmq_hbm, vpad_hbm, grow_hbm, vpg_hbm,
              vidx_v, idx_v, rows_v, vrows_v, vsem, sem):
        wid = lax.axis_index("s") * info.num_cores + lax.axis_index("c")
        base = wid * rpw
        pltpu.sync_copy(pc_hbm.at[pl.ds(base, rpw)], vidx_v)
        cpv = pltpu.async_copy(vpad_hbm.at[vidx_v], vrows_v, vsem)
        for ch in range(nch):
            pltpu.sync_copy(pcq_hbm.at[pl.ds(base + ch * qtr, qtr)],
                            idx_v.at[ch])
        cps = [None] * nch
        for ch in range(2):
            cps[ch] = pltpu.async_copy(gmq_hbm.at[idx_v.at[ch]],
                                       rows_v.at[ch % 2], sem)
        for ch in range(nch):
            cps[ch].wait()
            pltpu.sync_copy(rows_v.at[ch % 2],
                            grow_hbm.at[pl.ds(base + ch * qtr, qtr)])
            if ch + 2 < nch:
                cps[ch + 2] = pltpu.async_copy(gmq_hbm.at[idx_v.at[ch + 2]],
                                               rows_v.at[ch % 2], sem)
        cpv.wait()
        pltpu.sync_copy(vrows_v, vpg_hbm.at[pl.ds(base, rpw)])

    return sc_fn(pc, pcq, gmq, vpad)


def _tc_loss(pc_row, psh_col, vpg, grow):
    def tc_fn(pc_ref, psh_ref, vp_ref, g_ref, out_ref, acc_ref):
        kt = pl.program_id(0)

        @pl.when(kt == 0)
        def _():
            acc_ref[...] = jnp.zeros((P, P), jnp.float32)

        w = g_ref[...]
        ext = (w >> psh_ref[...]) & 1
        kio = lax.broadcasted_iota(jnp.int32, (KT, P), 0) + kt * KT
        oh = (kio == pc_ref[...]).astype(jnp.bfloat16)
        acc_ref[...] += jnp.dot(ext.astype(jnp.bfloat16), oh,
                                preferred_element_type=jnp.float32)

        @pl.when(kt == NKT - 1)
        def _():
            vp = vp_ref[...]
            s = jnp.zeros((P, P), jnp.float32)
            for c in range(3):
                col = vp[:, c:c + 1]
                e = (lax.broadcasted_iota(jnp.int32, (1, VW), 1) == c
                     ).astype(jnp.float32)
                row = lax.dot_general(e, vp, (((1,), (1,)), ((), ())),
                                      preferred_element_type=jnp.float32)
                d = col - row
                s = s + d * d
            big = jnp.float32(3.0e37)
            sm = jnp.where(acc_ref[...] > 0.5, s, big)
            rmin = jnp.min(sm, axis=1, keepdims=True)
            rmin = jnp.where(rmin >= big * 0.5, s[:, 0:1], rmin)
            out_ref[0, 0] = jnp.mean(jnp.tanh(jnp.sqrt(rmin)))

    out = pl.pallas_call(
        tc_fn,
        grid=(NKT,),
        in_specs=[
            pl.BlockSpec((1, P), lambda kt: (0, 0)),
            pl.BlockSpec((P, 1), lambda kt: (0, 0)),
            pl.BlockSpec((P, VW), lambda kt: (0, 0)),
            pl.BlockSpec((P, KT), lambda kt: (0, kt)),
        ],
        out_specs=pl.BlockSpec(memory_space=pltpu.SMEM),
        out_shape=jax.ShapeDtypeStruct((1, 1), jnp.float32),
        scratch_shapes=[pltpu.VMEM((P, P), jnp.float32)],
    )(pc_row, psh_col, vpg, grow)
    return out[0, 0]


def kernel(presented_contact, vertices, geomask):
    pc = presented_contact.astype(jnp.int32)
    gm8 = geomask.view(jnp.uint8)
    gmq, vpad = _repack(gm8, vertices[0])
    grow, vpg = _sc_gather(pc, pc >> 2, gmq, vpad)
    psh = ((pc & 3) * 8).reshape(P, 1)
    return _tc_loss(pc.reshape(1, P), psh, vpg, grow)


# speedup vs baseline: 55.6263x; 1.0088x over previous
import functools

import jax
import jax.numpy as jnp
from jax import lax
from jax.experimental import pallas as pl
from jax.experimental.pallas import tpu as pltpu
from jax.experimental.pallas import tpu_sc as plsc

NV = 6890
P = 1024
NVP = 6912
NQ = NVP // 4
VW = 128
KT = 2304
NKT = NVP // KT
WRB = 96
NWRB = NQ // WRB


def _repack(gm, v):
    def rp_fn(gm_ref, v_ref, gmq_ref, vp_ref):
        x = gm_ref[...]
        xp = jnp.pad(x, ((0, 0), (0, NVP - NV)))
        gmq_ref[...] = pltpu.bitcast(xp, jnp.int32)

        @pl.when(pl.program_id(0) == 0)
        def _():
            vp_ref[...] = jnp.pad(v_ref[...], ((0, 0), (0, VW - 3)))

    return pl.pallas_call(
        rp_fn,
        grid=(NWRB,),
        in_specs=[
            pl.BlockSpec((4 * WRB, NV), lambda r: (r, 0)),
            pl.BlockSpec((NV, 3), lambda r: (0, 0)),
        ],
        out_specs=[
            pl.BlockSpec((WRB, NVP), lambda r: (r, 0)),
            pl.BlockSpec((NV, VW), lambda r: (0, 0)),
        ],
        out_shape=[
            jax.ShapeDtypeStruct((NQ, NVP), jnp.int32),
            jax.ShapeDtypeStruct((NV, VW), jnp.float32),
        ],
    )(gm, v)


def _sc_gather(pc, pcq, gmq, vpad):
    info = plsc.get_sparse_core_info()
    nw = info.num_cores * info.num_subcores
    rpw = P // nw
    qtr = rpw // 4
    nch = 4

    mesh = plsc.VectorSubcoreMesh(core_axis_name="c", subcore_axis_name="s")

    @functools.partial(
        pl.kernel,
        mesh=mesh,
        out_type=[
            jax.ShapeDtypeStruct((P, NVP), jnp.int32),
            jax.ShapeDtypeStruct((P, VW), jnp.float32),
        ],
        scratch_types=[
            pltpu.VMEM((rpw,), jnp.int32),
            pltpu.VMEM((nch, qtr), jnp.int32),
            pltpu.VMEM((2, qtr, NVP), jnp.int32),
            pltpu.VMEM((rpw, VW), jnp.float32),
            pltpu.SemaphoreType.DMA,
            pltpu.SemaphoreType.DMA,
        ],
    )
    def sc_fn(pc_hbm, pcq_hbm, gmq_hbm, vpad_hbm, grow_hbm, vpg_hbm,
              vidx_v, idx_v, rows_v, vrows_v, vsem, sem):
        wid = lax.axis_index("s") * info.num_cores + lax.axis_index("c")
        base = wid * rpw
        pltpu.sync_copy(pc_hbm.at[pl.ds(base, rpw)], vidx_v)
        cpv = pltpu.async_copy(vpad_hbm.at[vidx_v], vrows_v, vsem)
        for ch in range(nch):
            pltpu.sync_copy(pcq_hbm.at[pl.ds(base + ch * qtr, qtr)],
                            idx_v.at[ch])
        cps = [None] * nch
        for ch in range(2):
            cps[ch] = pltpu.async_copy(gmq_hbm.at[idx_v.at[ch]],
                                       rows_v.at[ch % 2], sem)
        for ch in range(nch):
            cps[ch].wait()
            pltpu.sync_copy(rows_v.at[ch % 2],
                            grow_hbm.at[pl.ds(base + ch * qtr, qtr)])
            if ch + 2 < nch:
                cps[ch + 2] = pltpu.async_copy(gmq_hbm.at[idx_v.at[ch + 2]],
                                               rows_v.at[ch % 2], sem)
        cpv.wait()
        pltpu.sync_copy(vrows_v, vpg_hbm.at[pl.ds(base, rpw)])

    return sc_fn(pc, pcq, gmq, vpad)


def _tc_loss(pc_row, psh_col, vpg, grow):
    def tc_fn(pc_ref, psh_ref, vp_ref, g_ref, out_ref, acc_ref):
        kt = pl.program_id(0)

        @pl.when(kt == 0)
        def _():
            acc_ref[...] = jnp.zeros((P, P), jnp.float32)

        w = g_ref[...]
        ext = (w >> psh_ref[...]) & 1
        kio = lax.broadcasted_iota(jnp.int32, (KT, P), 0) + kt * KT
        oh = (kio == pc_ref[...]).astype(jnp.bfloat16)
        acc_ref[...] += jnp.dot(ext.astype(jnp.bfloat16), oh,
                                preferred_element_type=jnp.float32)

        @pl.when(kt == NKT - 1)
        def _():
            vp = vp_ref[...]
            s = jnp.zeros((P, P), jnp.float32)
            for c in range(3):
                col = vp[:, c:c + 1]
                e = (lax.broadcasted_iota(jnp.int32, (1, VW), 1) == c
                     ).astype(jnp.float32)
                row = lax.dot_general(e, vp, (((1,), (1,)), ((), ())),
                                      preferred_element_type=jnp.float32)
                d = col - row
                s = s + d * d
            big = jnp.float32(3.0e37)
            sm = jnp.where(acc_ref[...] > 0.5, s, big)
            rmin = jnp.min(sm, axis=1, keepdims=True)
            rmin = jnp.where(rmin >= big * 0.5, s[:, 0:1], rmin)
            out_ref[0, 0] = jnp.mean(jnp.tanh(jnp.sqrt(rmin)))

    out = pl.pallas_call(
        tc_fn,
        grid=(NKT,),
        in_specs=[
            pl.BlockSpec((1, P), lambda kt: (0, 0)),
            pl.BlockSpec((P, 1), lambda kt: (0, 0)),
            pl.BlockSpec((P, VW), lambda kt: (0, 0)),
            pl.BlockSpec((P, KT), lambda kt: (0, kt)),
        ],
        out_specs=pl.BlockSpec(memory_space=pltpu.SMEM),
        out_shape=jax.ShapeDtypeStruct((1, 1), jnp.float32),
        scratch_shapes=[pltpu.VMEM((P, P), jnp.float32)],
    )(pc_row, psh_col, vpg, grow)
    return out[0, 0]


def kernel(presented_contact, vertices, geomask):
    pc = presented_contact.astype(jnp.int32)
    gm8 = geomask.view(jnp.uint8)
    gmq, vpad = _repack(gm8, vertices[0])
    grow, vpg = _sc_gather(pc, pc >> 2, gmq, vpad)
    psh = ((pc & 3) * 8).reshape(P, 1)
    return _tc_loss(pc.reshape(1, P), psh, vpg, grow)
